# Initial kernel scaffold; baseline (speedup 1.0000x reference)
import jax
import jax.numpy as jnp
from jax.experimental import pallas as pl


def kernel(section_scores, video_scores, labels, segments):
    raise NotImplementedError("write your pallas kernel here")


# speedup vs baseline: 1.2918x; 1.2918x over previous
import functools

import jax
import jax.numpy as jnp
from jax.experimental import pallas as pl
from jax.experimental.pallas import tpu as pltpu

_V = 256


def _body(x_ref, v_ref, y_ref, out_ref):
    i = pl.program_id(0)
    x = x_ref[...]
    pooled = jnp.max(x, axis=1)
    y = y_ref[...]
    v = v_ref[...]

    def bce_sum(p):
        logp = jnp.maximum(jnp.log(p), -100.0)
        log1mp = jnp.maximum(jnp.log1p(-p), -100.0)
        return jnp.sum(y * logp + (1.0 - y) * log1mp)

    s = bce_sum(v) + bce_sum(pooled)

    @pl.when(i == 0)
    def _():
        out_ref[0, 0] = 0.0

    out_ref[0, 0] += s


@jax.jit
def kernel(section_scores, video_scores, labels, segments):
    b, s = segments.shape
    c = section_scores.shape[1]
    x3 = section_scores.reshape(b, s, c)
    grid = b // _V
    acc = pl.pallas_call(
        _body,
        grid=(grid,),
        in_specs=[
            pl.BlockSpec((_V, s, c), lambda i: (i, 0, 0)),
            pl.BlockSpec((_V, c), lambda i: (i, 0)),
            pl.BlockSpec((_V, c), lambda i: (i, 0)),
        ],
        out_specs=pl.BlockSpec((1, 1), lambda i: (0, 0), memory_space=pltpu.SMEM),
        out_shape=jax.ShapeDtypeStruct((1, 1), jnp.float32),
    )(x3, video_scores, labels)
    return -acc[0, 0] / (b * c)


# speedup vs baseline: 4.2039x; 3.2542x over previous
import functools

import jax
import jax.numpy as jnp
from jax.experimental import pallas as pl
from jax.experimental.pallas import tpu as pltpu

_V = 256


def _body(s, x_ref, v_ref, y_ref, out_ref):
    i = pl.program_id(0)
    x = x_ref[...]
    pooled = jnp.max(x.reshape(_V, s, x.shape[-1]), axis=1)
    y = y_ref[...]
    v = v_ref[...]

    def bce_sum(p):
        logp = jnp.maximum(jnp.log(p), -100.0)
        log1mp = jnp.maximum(jnp.log1p(-p), -100.0)
        return jnp.sum(y * logp + (1.0 - y) * log1mp)

    s = bce_sum(v) + bce_sum(pooled)

    @pl.when(i == 0)
    def _():
        out_ref[0, 0] = 0.0

    out_ref[0, 0] += s


@jax.jit
def kernel(section_scores, video_scores, labels, segments):
    b, s = segments.shape
    c = section_scores.shape[1]
    grid = b // _V
    acc = pl.pallas_call(
        functools.partial(_body, s),
        grid=(grid,),
        in_specs=[
            pl.BlockSpec((_V * s, c), lambda i: (i, 0)),
            pl.BlockSpec((_V, c), lambda i: (i, 0)),
            pl.BlockSpec((_V, c), lambda i: (i, 0)),
        ],
        out_specs=pl.BlockSpec((1, 1), lambda i: (0, 0), memory_space=pltpu.SMEM),
        out_shape=jax.ShapeDtypeStruct((1, 1), jnp.float32),
    )(section_scores, video_scores, labels)
    return -acc[0, 0] / (b * c)
